# baseline (speedup 1.0000x reference)
import jax
import jax.numpy as jnp
from jax.experimental import pallas as pl
from jax.experimental.pallas import tpu as pltpu

_EPS = 1e-6
_LOG2E = 1.4426950408889634


def _main_body(x_ref, cond_ref, w_ada_ref, b_ada_ref, w_proj_ref,
               b_proj_ref, out_ref):
    d = x_ref.shape[-1]
    inv_d = 1.0 / d
    c = cond_ref[0]
    cs = c * jax.nn.sigmoid(c)
    mod = (
        jnp.dot(cs, w_ada_ref[...], preferred_element_type=jnp.float32)
        + b_ada_ref[...]
    )
    a_b = (1.0 + mod[:, :d]).astype(jnp.bfloat16)
    b_b = mod[:, d:].astype(jnp.bfloat16)

    x = x_ref[0]
    xb = x.astype(jnp.bfloat16)
    s1 = jnp.sum(xb, axis=-1, keepdims=True, dtype=jnp.bfloat16)
    s2 = jnp.sum(xb * xb, axis=-1, keepdims=True, dtype=jnp.bfloat16)
    mu = s1.astype(jnp.float32) * inv_d
    var = s2.astype(jnp.float32) * inv_d - mu * mu
    r = jax.lax.rsqrt(var + _EPS)
    xn = (x - mu) * r
    z = xn.astype(jnp.bfloat16) * a_b + b_b
    y = z / (1.0 + jnp.exp2(z * jnp.bfloat16(-_LOG2E)))
    out_ref[0] = (
        jnp.dot(y, w_proj_ref[...], preferred_element_type=jnp.float32)
        + b_proj_ref[...]
    )


def kernel(x, cond, w_ada, b_ada, w_proj, b_proj):
    B, T, D = x.shape
    OUT = w_proj.shape[1]
    TBLK = 4096

    grid = (B, T // TBLK)
    out = pl.pallas_call(
        _main_body,
        out_shape=jax.ShapeDtypeStruct((B, T, OUT), jnp.float32),
        grid=grid,
        in_specs=[
            pl.BlockSpec((1, TBLK, D), lambda b, t: (b, t, 0)),
            pl.BlockSpec((1, 1, D), lambda b, t: (b, 0, 0)),
            pl.BlockSpec((D, 2 * D), lambda b, t: (0, 0)),
            pl.BlockSpec((1, 2 * D), lambda b, t: (0, 0)),
            pl.BlockSpec((D, OUT), lambda b, t: (0, 0)),
            pl.BlockSpec((1, OUT), lambda b, t: (0, 0)),
        ],
        out_specs=pl.BlockSpec((1, TBLK, OUT), lambda b, t: (b, t, 0)),
        compiler_params=pltpu.CompilerParams(
            dimension_semantics=("parallel", "arbitrary"),
            vmem_limit_bytes=56 * 1024 * 1024,
        ),
    )(x, cond.reshape(B, 1, D), w_ada, b_ada.reshape(1, 2 * D),
      w_proj.astype(jnp.bfloat16), b_proj.reshape(1, OUT))
    return out
